# baseline (speedup 1.0000x reference)
import functools

import jax
import jax.numpy as jnp
from jax import lax
from jax.experimental import pallas as pl
from jax.experimental.pallas import tpu as pltpu
from jax.experimental.pallas import tpu_sc as plsc

NC = 2
NS = 16
NW = NC * NS

CHUNK = 1600


def _tc_table_pack(wt, vocab):
    cols = 16384
    sub = cols // 8
    rep = 4
    grid = (vocab + cols * rep - 1) // (cols * rep)

    def body(w_ref, o_ref):
        for h in range(rep):
            u16 = lax.bitcast_convert_type(
                w_ref[:, h * cols : (h + 1) * cols].astype(jnp.bfloat16),
                jnp.uint16,
            ).astype(jnp.uint32)
            u3 = u16.reshape(16, 2, cols)
            packed = u3[:, 0, :] | (u3[:, 1, :] << 16)
            b = jnp.concatenate(
                [packed[:, q * sub : (q + 1) * sub] for q in range(8)],
                axis=0,
            )
            o_ref[h * sub : (h + 1) * sub, :] = lax.bitcast_convert_type(
                b.T, jnp.int32
            )

    return pl.pallas_call(
        body,
        grid=grid,
        in_specs=[pl.BlockSpec((32, cols * rep), lambda i: (0, i))],
        out_specs=pl.BlockSpec((sub * rep, 128), lambda i: (i, 0)),
        out_shape=jax.ShapeDtypeStruct((grid * sub * rep, 128), jnp.int32),
    )(wt)


def _sc_gather(n_rows, x_flat, table):
    b_per_w = n_rows // NW
    n_chunks = b_per_w // CHUNK
    mesh = plsc.VectorSubcoreMesh(core_axis_name="c", subcore_axis_name="s")

    @functools.partial(
        pl.kernel,
        out_type=jax.ShapeDtypeStruct((n_rows, 16), jnp.int32),
        mesh=mesh,
        scratch_types=[
            pltpu.VMEM((b_per_w,), jnp.int32),
            pltpu.VMEM((CHUNK, 16), jnp.int32),
            pltpu.VMEM((CHUNK, 16), jnp.int32),
            pltpu.SemaphoreType.DMA,
            pltpu.SemaphoreType.DMA,
            pltpu.SemaphoreType.DMA,
            pltpu.SemaphoreType.DMA,
        ],
        compiler_params=pltpu.CompilerParams(use_tc_tiling_on_sc=False),
    )
    def body(table_hbm, idx_hbm, out_hbm, idx_v, rows0, rows1, g0, g1, o0, o1):
        wid = lax.axis_index("s") * NC + lax.axis_index("c")
        base = wid * b_per_w
        rows = (rows0, rows1)
        gsem = (g0, g1)
        osem = (o0, o1)

        pltpu.sync_copy(idx_hbm.at[pl.ds(base, b_per_w)], idx_v)

        def start_gather(i):
            return pltpu.async_copy(
                table_hbm.at[idx_v.at[pl.ds(i * CHUNK, CHUNK)]],
                rows[i % 2],
                gsem[i % 2],
            )

        def start_out(i):
            return pltpu.async_copy(
                rows[i % 2],
                out_hbm.at[pl.ds(base + i * CHUNK, CHUNK)],
                osem[i % 2],
            )

        gathers = [None] * n_chunks
        outs = [None] * n_chunks
        gathers[0] = start_gather(0)
        for i in range(n_chunks):
            gathers[i].wait()
            if i + 1 < n_chunks:
                if i >= 1:
                    outs[i - 1].wait()
                gathers[i + 1] = start_gather(i + 1)
            outs[i] = start_out(i)
        outs[n_chunks - 1].wait()
        if n_chunks >= 2:
            outs[n_chunks - 2].wait()

    return body(table, x_flat)


def _tc_out_fixup(g3, l_count, b_count, l_off, l_total, p_prev=None):
    n_t = b_count // 1024
    rep = 5
    blk_off = l_off // rep

    def body(*refs):
        g_ref = refs[0]
        p_ref = refs[-1]
        for r in range(rep):
            pieces = []
            for t in range(n_t):
                w = g_ref[r, 128 * t : 128 * (t + 1), :].T
                wu = lax.bitcast_convert_type(w, jnp.uint32)
                f_lo = lax.bitcast_convert_type(wu << 16, jnp.float32)
                f_hi = lax.bitcast_convert_type(
                    (wu >> 16) << 16, jnp.float32
                )
                for a in range(8):
                    lo = f_lo[16 * a : 16 * (a + 1), :]
                    hi = f_hi[16 * a : 16 * (a + 1), :]
                    pieces.append(
                        jnp.stack([lo, hi], axis=1).reshape(32, 128)
                    )
            p_ref[r] = jnp.concatenate(pieces, axis=1)

    in_specs = [pl.BlockSpec((rep, b_count // 8, 128), lambda i: (i, 0, 0))]
    args = [g3]
    aliases = {}
    if p_prev is not None:
        in_specs.append(pl.BlockSpec(memory_space=pl.ANY))
        args.append(p_prev)
        aliases = {1: 0}

    return pl.pallas_call(
        body,
        grid=(l_count // rep,),
        in_specs=in_specs,
        out_specs=pl.BlockSpec(
            (rep, 32, b_count), lambda i: (i + blk_off, 0, 0)
        ),
        out_shape=jax.ShapeDtypeStruct((l_total, 32, b_count), jnp.float32),
        input_output_aliases=aliases,
    )(*args)


def kernel(x, weight):
    b, l = x.shape
    vocab, emb = weight.shape

    wlin = _tc_table_pack(weight.T, vocab)
    table = wlin.reshape(wlin.shape[0] * 8, 16)

    xi = x.astype(jnp.int32)
    xv = (xi // 16384) * 16384 + 8 * (xi % 2048) + (xi % 16384) // 2048
    xp = (
        xv.reshape(b // 1024, 8, 128, l)
        .transpose(0, 2, 1, 3)
        .reshape(b, l)
        .T.reshape(-1)
    )

    n_sl = 8
    l_sl = l // n_sl
    rows_sl = l_sl * b
    p = None
    for h in range(n_sl):
        g = _sc_gather(
            rows_sl,
            lax.slice(xp, (h * rows_sl,), ((h + 1) * rows_sl,)),
            table,
        )
        g3 = g.reshape(l_sl, b // 8, 128)
        p = _tc_out_fixup(g3, l_sl, b, h * l_sl, l, p)
    return p.transpose(2, 0, 1)


# speedup vs baseline: 1.0320x; 1.0320x over previous
import functools

import jax
import jax.numpy as jnp
from jax import lax
from jax.experimental import pallas as pl
from jax.experimental.pallas import tpu as pltpu
from jax.experimental.pallas import tpu_sc as plsc

NC = 2
NS = 16
NW = NC * NS

CHUNK = 1600


def _tc_table_pack(wt, vocab):
    cols = 16384
    sub = cols // 8
    rep = 4
    grid = (vocab + cols * rep - 1) // (cols * rep)

    def body(w_ref, o_ref):
        for h in range(rep):
            u16 = lax.bitcast_convert_type(
                w_ref[:, h * cols : (h + 1) * cols].astype(jnp.bfloat16),
                jnp.uint16,
            ).astype(jnp.uint32)
            u3 = u16.reshape(16, 2, cols)
            packed = u3[:, 0, :] | (u3[:, 1, :] << 16)
            b = jnp.concatenate(
                [packed[:, q * sub : (q + 1) * sub] for q in range(8)],
                axis=0,
            )
            o_ref[h * sub : (h + 1) * sub, :] = lax.bitcast_convert_type(
                b.T, jnp.int32
            )

    return pl.pallas_call(
        body,
        grid=grid,
        in_specs=[pl.BlockSpec((32, cols * rep), lambda i: (0, i))],
        out_specs=pl.BlockSpec((sub * rep, 128), lambda i: (i, 0)),
        out_shape=jax.ShapeDtypeStruct((grid * sub * rep, 128), jnp.int32),
    )(wt)


def _sc_gather(n_rows, x_flat, table):
    b_per_w = n_rows // NW
    n_chunks = b_per_w // CHUNK
    mesh = plsc.VectorSubcoreMesh(core_axis_name="c", subcore_axis_name="s")

    @functools.partial(
        pl.kernel,
        out_type=jax.ShapeDtypeStruct((n_rows, 16), jnp.int32),
        mesh=mesh,
        scratch_types=[
            pltpu.VMEM((b_per_w,), jnp.int32),
            pltpu.VMEM((CHUNK, 16), jnp.int32),
            pltpu.VMEM((CHUNK, 16), jnp.int32),
            pltpu.SemaphoreType.DMA,
            pltpu.SemaphoreType.DMA,
            pltpu.SemaphoreType.DMA,
            pltpu.SemaphoreType.DMA,
        ],
        compiler_params=pltpu.CompilerParams(use_tc_tiling_on_sc=False),
    )
    def body(table_hbm, idx_hbm, out_hbm, idx_v, rows0, rows1, g0, g1, o0, o1):
        wid = lax.axis_index("s") * NC + lax.axis_index("c")
        base = wid * b_per_w
        rows = (rows0, rows1)
        gsem = (g0, g1)
        osem = (o0, o1)

        pltpu.sync_copy(idx_hbm.at[pl.ds(base, b_per_w)], idx_v)

        def start_gather(i):
            return pltpu.async_copy(
                table_hbm.at[idx_v.at[pl.ds(i * CHUNK, CHUNK)]],
                rows[i % 2],
                gsem[i % 2],
            )

        def start_out(i):
            return pltpu.async_copy(
                rows[i % 2],
                out_hbm.at[pl.ds(base + i * CHUNK, CHUNK)],
                osem[i % 2],
            )

        gathers = [None] * n_chunks
        outs = [None] * n_chunks
        gathers[0] = start_gather(0)
        for i in range(n_chunks):
            gathers[i].wait()
            if i + 1 < n_chunks:
                if i >= 1:
                    outs[i - 1].wait()
                gathers[i + 1] = start_gather(i + 1)
            outs[i] = start_out(i)
        outs[n_chunks - 1].wait()
        if n_chunks >= 2:
            outs[n_chunks - 2].wait()

    return body(table, x_flat)


def _tc_out_fixup(g3, l_count, b_count, l_off, l_total, p_prev=None):
    n_t = b_count // 1024
    rep = 5
    blk_off = l_off // rep

    def body(*refs):
        g_ref = refs[0]
        p_ref = refs[-1]
        for r in range(rep):
            pieces = []
            for t in range(n_t):
                w = g_ref[r, 128 * t : 128 * (t + 1), :].T
                wu = lax.bitcast_convert_type(w, jnp.uint32)
                f_lo = lax.bitcast_convert_type(wu << 16, jnp.float32)
                f_hi = lax.bitcast_convert_type(
                    (wu >> 16) << 16, jnp.float32
                )
                for a in range(8):
                    lo = f_lo[16 * a : 16 * (a + 1), :]
                    hi = f_hi[16 * a : 16 * (a + 1), :]
                    pieces.append(
                        jnp.stack([lo, hi], axis=1).reshape(32, 128)
                    )
            p_ref[r] = jnp.concatenate(pieces, axis=1)

    in_specs = [pl.BlockSpec((rep, b_count // 8, 128), lambda i: (i, 0, 0))]
    args = [g3]
    aliases = {}
    if p_prev is not None:
        in_specs.append(pl.BlockSpec(memory_space=pl.ANY))
        args.append(p_prev)
        aliases = {1: 0}

    return pl.pallas_call(
        body,
        grid=(l_count // rep,),
        in_specs=in_specs,
        out_specs=pl.BlockSpec(
            (rep, 32, b_count), lambda i: (i + blk_off, 0, 0)
        ),
        out_shape=jax.ShapeDtypeStruct((l_total, 32, b_count), jnp.float32),
        input_output_aliases=aliases,
    )(*args)


def kernel(x, weight):
    b, l = x.shape
    vocab, emb = weight.shape

    wlin = _tc_table_pack(weight.T, vocab)
    table = wlin.reshape(wlin.shape[0] * 8, 16)

    xi = x.astype(jnp.int32)
    xv = (xi // 16384) * 16384 + 8 * (xi % 2048) + (xi % 16384) // 2048
    xp = (
        xv.reshape(b // 1024, 8, 128, l)
        .transpose(0, 2, 1, 3)
        .reshape(b, l)
        .T.reshape(-1)
    )

    n_sl = 4
    l_sl = l // n_sl
    rows_sl = l_sl * b
    p = None
    for h in range(n_sl):
        g = _sc_gather(
            rows_sl,
            lax.slice(xp, (h * rows_sl,), ((h + 1) * rows_sl,)),
            table,
        )
        g3 = g.reshape(l_sl, b // 8, 128)
        p = _tc_out_fixup(g3, l_sl, b, h * l_sl, l, p)
    return p.transpose(2, 0, 1)


# speedup vs baseline: 1.0407x; 1.0085x over previous
import functools

import jax
import jax.numpy as jnp
from jax import lax
from jax.experimental import pallas as pl
from jax.experimental.pallas import tpu as pltpu
from jax.experimental.pallas import tpu_sc as plsc

NC = 2
NS = 16
NW = NC * NS

CHUNK = 1600


def _tc_table_pack(wt, vocab):
    cols = 16384
    sub = cols // 8
    rep = 4
    grid = (vocab + cols * rep - 1) // (cols * rep)

    def body(w_ref, o_ref):
        for h in range(rep):
            u16 = lax.bitcast_convert_type(
                w_ref[:, h * cols : (h + 1) * cols].astype(jnp.bfloat16),
                jnp.uint16,
            ).astype(jnp.uint32)
            u3 = u16.reshape(16, 2, cols)
            packed = u3[:, 0, :] | (u3[:, 1, :] << 16)
            b = jnp.concatenate(
                [packed[:, q * sub : (q + 1) * sub] for q in range(8)],
                axis=0,
            )
            o_ref[h * sub : (h + 1) * sub, :] = lax.bitcast_convert_type(
                b.T, jnp.int32
            )

    return pl.pallas_call(
        body,
        grid=grid,
        in_specs=[pl.BlockSpec((32, cols * rep), lambda i: (0, i))],
        out_specs=pl.BlockSpec((sub * rep, 128), lambda i: (i, 0)),
        out_shape=jax.ShapeDtypeStruct((grid * sub * rep, 128), jnp.int32),
    )(wt)


def _sc_gather(n_rows, x_flat, table):
    b_per_w = n_rows // NW
    n_chunks = b_per_w // CHUNK
    mesh = plsc.VectorSubcoreMesh(core_axis_name="c", subcore_axis_name="s")

    @functools.partial(
        pl.kernel,
        out_type=jax.ShapeDtypeStruct((n_rows, 16), jnp.int32),
        mesh=mesh,
        scratch_types=[
            pltpu.VMEM((b_per_w,), jnp.int32),
            pltpu.VMEM((CHUNK, 16), jnp.int32),
            pltpu.VMEM((CHUNK, 16), jnp.int32),
            pltpu.SemaphoreType.DMA,
            pltpu.SemaphoreType.DMA,
            pltpu.SemaphoreType.DMA,
            pltpu.SemaphoreType.DMA,
        ],
        compiler_params=pltpu.CompilerParams(use_tc_tiling_on_sc=False),
    )
    def body(table_hbm, idx_hbm, out_hbm, idx_v, rows0, rows1, g0, g1, o0, o1):
        wid = lax.axis_index("s") * NC + lax.axis_index("c")
        base = wid * b_per_w
        rows = (rows0, rows1)
        gsem = (g0, g1)
        osem = (o0, o1)

        pltpu.sync_copy(idx_hbm.at[pl.ds(base, b_per_w)], idx_v)

        def start_gather(i):
            return pltpu.async_copy(
                table_hbm.at[idx_v.at[pl.ds(i * CHUNK, CHUNK)]],
                rows[i % 2],
                gsem[i % 2],
            )

        def start_out(i):
            return pltpu.async_copy(
                rows[i % 2],
                out_hbm.at[pl.ds(base + i * CHUNK, CHUNK)],
                osem[i % 2],
            )

        gathers = [None] * n_chunks
        outs = [None] * n_chunks
        gathers[0] = start_gather(0)
        for i in range(n_chunks):
            gathers[i].wait()
            if i + 1 < n_chunks:
                if i >= 1:
                    outs[i - 1].wait()
                gathers[i + 1] = start_gather(i + 1)
            outs[i] = start_out(i)
        outs[n_chunks - 1].wait()
        if n_chunks >= 2:
            outs[n_chunks - 2].wait()

    return body(table, x_flat)


def _tc_out_fixup(g3, l_count, b_count, l_off, l_total, p_prev=None):
    n_t = b_count // 1024
    rep = 10
    blk_off = l_off // rep

    def body(*refs):
        g_ref = refs[0]
        p_ref = refs[-1]
        for r in range(rep):
            pieces = []
            for t in range(n_t):
                w = g_ref[r, 128 * t : 128 * (t + 1), :].T
                wu = lax.bitcast_convert_type(w, jnp.uint32)
                f_lo = lax.bitcast_convert_type(wu << 16, jnp.float32)
                f_hi = lax.bitcast_convert_type(
                    (wu >> 16) << 16, jnp.float32
                )
                for a in range(8):
                    lo = f_lo[16 * a : 16 * (a + 1), :]
                    hi = f_hi[16 * a : 16 * (a + 1), :]
                    pieces.append(
                        jnp.stack([lo, hi], axis=1).reshape(32, 128)
                    )
            p_ref[r] = jnp.concatenate(pieces, axis=1)

    in_specs = [pl.BlockSpec((rep, b_count // 8, 128), lambda i: (i, 0, 0))]
    args = [g3]
    aliases = {}
    if p_prev is not None:
        in_specs.append(pl.BlockSpec(memory_space=pl.ANY))
        args.append(p_prev)
        aliases = {1: 0}

    return pl.pallas_call(
        body,
        grid=(l_count // rep,),
        in_specs=in_specs,
        out_specs=pl.BlockSpec(
            (rep, 32, b_count), lambda i: (i + blk_off, 0, 0)
        ),
        out_shape=jax.ShapeDtypeStruct((l_total, 32, b_count), jnp.float32),
        input_output_aliases=aliases,
    )(*args)


def kernel(x, weight):
    b, l = x.shape
    vocab, emb = weight.shape

    wlin = _tc_table_pack(weight.T, vocab)
    table = wlin.reshape(wlin.shape[0] * 8, 16)

    xi = x.astype(jnp.int32)
    xv = (xi // 16384) * 16384 + 8 * (xi % 2048) + (xi % 16384) // 2048
    xp = (
        xv.reshape(b // 1024, 8, 128, l)
        .transpose(0, 2, 1, 3)
        .reshape(b, l)
        .T.reshape(-1)
    )

    n_sl = 4
    l_sl = l // n_sl
    rows_sl = l_sl * b
    p = None
    for h in range(n_sl):
        g = _sc_gather(
            rows_sl,
            lax.slice(xp, (h * rows_sl,), ((h + 1) * rows_sl,)),
            table,
        )
        g3 = g.reshape(l_sl, b // 8, 128)
        p = _tc_out_fixup(g3, l_sl, b, h * l_sl, l, p)
    return p.transpose(2, 0, 1)
